# baseline (speedup 1.0000x reference)
import functools

import jax
import jax.numpy as jnp
from jax import lax
from jax.experimental import pallas as pl
from jax.experimental.pallas import tpu as pltpu
from jax.experimental.pallas import tpu_sc as plsc

BATCH = 4096
DIM = 3328
NUM_WORKERS = 32
ROWS_PER_WORKER = BATCH // NUM_WORKERS
CHUNK_ROWS = 2
NUM_CHUNKS = ROWS_PER_WORKER // CHUNK_ROWS
VREGS_PER_ROW = DIM // 16


def _sc_body(t0, t1, t2, t3, t4, t5, t6, t7, out, bufs, sem):
    tensors = (t0, t1, t2, t3, t4, t5, t6, t7)
    wid = lax.axis_index("s") * 2 + lax.axis_index("c")
    row0 = wid * ROWS_PER_WORKER

    def chunk_body(ci, _):
        r0 = row0 + ci * CHUNK_ROWS
        cps = [
            pltpu.async_copy(
                tensors[k].at[pl.ds(r0, CHUNK_ROWS), :],
                bufs.at[k], sem)
            for k in range(8)
        ]
        for cp in cps:
            cp.wait()

        def add_body(j, _):
            for r in range(CHUNK_ROWS):
                sl = pl.ds(j * 16, 16)
                s = ((bufs[0, r, sl] + bufs[1, r, sl])
                     + (bufs[2, r, sl] + bufs[3, r, sl])) + \
                    ((bufs[4, r, sl] + bufs[5, r, sl])
                     + (bufs[6, r, sl] + bufs[7, r, sl]))
                bufs[0, r, sl] = s
            return 0

        lax.fori_loop(0, VREGS_PER_ROW, add_body, 0, unroll=2)
        pltpu.sync_copy(bufs.at[0], out.at[pl.ds(r0, CHUNK_ROWS), :])
        return 0

    lax.fori_loop(0, NUM_CHUNKS, chunk_body, 0)


def kernel(tensors_0, tensors_1, tensors_2, tensors_3, tensors_4, tensors_5, tensors_6, tensors_7):
    mesh = plsc.VectorSubcoreMesh(core_axis_name="c", subcore_axis_name="s")
    k = functools.partial(
        pl.kernel,
        mesh=mesh,
        out_type=jax.ShapeDtypeStruct((BATCH, DIM), jnp.float32),
        scratch_types=[
            pltpu.VMEM((8, CHUNK_ROWS, DIM), jnp.float32),
            pltpu.SemaphoreType.DMA,
        ],
    )(_sc_body)
    return k(tensors_0, tensors_1, tensors_2, tensors_3,
             tensors_4, tensors_5, tensors_6, tensors_7)


# speedup vs baseline: 1.5907x; 1.5907x over previous
import functools

import jax
import jax.numpy as jnp
from jax import lax
from jax.experimental import pallas as pl
from jax.experimental.pallas import tpu as pltpu
from jax.experimental.pallas import tpu_sc as plsc

BATCH = 4096
DIM = 3328
NUM_WORKERS = 32
ROWS_PER_WORKER = BATCH // NUM_WORKERS
CHUNK_ROWS = 2
NUM_CHUNKS = ROWS_PER_WORKER // CHUNK_ROWS
VREGS_PER_ROW = DIM // 16


def _sc_body(t0, t1, t2, t3, t4, t5, t6, t7, out,
             bufs, obuf, in_sems, out_sems):
    tensors = (t0, t1, t2, t3, t4, t5, t6, t7)
    wid = lax.axis_index("s") * 2 + lax.axis_index("c")
    row0 = wid * ROWS_PER_WORKER

    def issue_in(c, b):
        r0 = row0 + c * CHUNK_ROWS
        for k in range(8):
            pltpu.async_copy(
                tensors[k].at[pl.ds(r0, CHUNK_ROWS), :],
                bufs.at[b, k], in_sems.at[b])

    def wait_in(b):
        for k in range(8):
            pltpu.make_async_copy(
                tensors[k].at[pl.ds(0, CHUNK_ROWS), :],
                bufs.at[b, k], in_sems.at[b]).wait()

    def wait_out(b):
        pltpu.make_async_copy(
            obuf.at[b], out.at[pl.ds(0, CHUNK_ROWS), :],
            out_sems.at[b]).wait()

    issue_in(0, 0)

    def outer(ci, _):
        for b in range(2):
            c = ci + b

            @pl.when(c + 1 < NUM_CHUNKS)
            def _():
                issue_in(c + 1, 1 - b)

            wait_in(b)

            @pl.when(c >= 2)
            def _():
                wait_out(b)

            def add_body(j, _):
                sl = pl.ds(j * 16, 16)
                for r in range(CHUNK_ROWS):
                    s = ((bufs[b, 0, r, sl] + bufs[b, 1, r, sl])
                         + (bufs[b, 2, r, sl] + bufs[b, 3, r, sl])) + \
                        ((bufs[b, 4, r, sl] + bufs[b, 5, r, sl])
                         + (bufs[b, 6, r, sl] + bufs[b, 7, r, sl]))
                    obuf[b, r, sl] = s
                return 0

            lax.fori_loop(0, VREGS_PER_ROW, add_body, 0, unroll=2)

            r0 = row0 + c * CHUNK_ROWS
            pltpu.async_copy(
                obuf.at[b], out.at[pl.ds(r0, CHUNK_ROWS), :],
                out_sems.at[b])
        return 0

    lax.fori_loop(0, NUM_CHUNKS // 2, lambda i, x: outer(i * 2, x), 0)
    wait_out(0)
    wait_out(1)


def kernel(tensors_0, tensors_1, tensors_2, tensors_3, tensors_4, tensors_5, tensors_6, tensors_7):
    mesh = plsc.VectorSubcoreMesh(core_axis_name="c", subcore_axis_name="s")
    k = functools.partial(
        pl.kernel,
        mesh=mesh,
        out_type=jax.ShapeDtypeStruct((BATCH, DIM), jnp.float32),
        scratch_types=[
            pltpu.VMEM((2, 8, CHUNK_ROWS, DIM), jnp.float32),
            pltpu.VMEM((2, CHUNK_ROWS, DIM), jnp.float32),
            pltpu.SemaphoreType.DMA((2,)),
            pltpu.SemaphoreType.DMA((2,)),
        ],
    )(_sc_body)
    return k(tensors_0, tensors_1, tensors_2, tensors_3,
             tensors_4, tensors_5, tensors_6, tensors_7)


# speedup vs baseline: 2.4804x; 1.5593x over previous
import functools

import jax
import jax.numpy as jnp
from jax import lax
from jax.experimental import pallas as pl
from jax.experimental.pallas import tpu as pltpu
from jax.experimental.pallas import tpu_sc as plsc

BATCH = 4096
DIM = 3328

SC_ROWS = 1280
TC_ROWS = BATCH - SC_ROWS

NUM_WORKERS = 32
ROWS_PER_WORKER = SC_ROWS // NUM_WORKERS
CHUNK_ROWS = 2
NUM_CHUNKS = ROWS_PER_WORKER // CHUNK_ROWS
VREGS_PER_ROW = DIM // 16

TC_BLOCK_ROWS = 256


def _sc_body(t0, t1, t2, t3, t4, t5, t6, t7, out,
             bufs, obuf, in_sems, out_sems):
    tensors = (t0, t1, t2, t3, t4, t5, t6, t7)
    wid = lax.axis_index("s") * 2 + lax.axis_index("c")
    row0 = wid * ROWS_PER_WORKER

    def issue_in(c, b):
        r0 = row0 + c * CHUNK_ROWS
        for k in range(8):
            pltpu.async_copy(
                tensors[k].at[pl.ds(r0, CHUNK_ROWS), :],
                bufs.at[b, k], in_sems.at[b])

    def wait_in(b):
        for k in range(8):
            pltpu.make_async_copy(
                tensors[k].at[pl.ds(0, CHUNK_ROWS), :],
                bufs.at[b, k], in_sems.at[b]).wait()

    def wait_out(b):
        pltpu.make_async_copy(
            obuf.at[b], out.at[pl.ds(0, CHUNK_ROWS), :],
            out_sems.at[b]).wait()

    issue_in(0, 0)

    def outer(ci, _):
        for b in range(2):
            c = ci + b

            @pl.when(c + 1 < NUM_CHUNKS)
            def _():
                issue_in(c + 1, 1 - b)

            wait_in(b)

            @pl.when(c >= 2)
            def _():
                wait_out(b)

            def add_body(j, _):
                sl = pl.ds(j * 16, 16)
                for r in range(CHUNK_ROWS):
                    s = ((bufs[b, 0, r, sl] + bufs[b, 1, r, sl])
                         + (bufs[b, 2, r, sl] + bufs[b, 3, r, sl])) + \
                        ((bufs[b, 4, r, sl] + bufs[b, 5, r, sl])
                         + (bufs[b, 6, r, sl] + bufs[b, 7, r, sl]))
                    obuf[b, r, sl] = s
                return 0

            lax.fori_loop(0, VREGS_PER_ROW, add_body, 0, unroll=2)

            r0 = row0 + c * CHUNK_ROWS
            pltpu.async_copy(
                obuf.at[b], out.at[pl.ds(r0, CHUNK_ROWS), :],
                out_sems.at[b])
        return 0

    lax.fori_loop(0, NUM_CHUNKS // 2, lambda i, x: outer(i * 2, x), 0)
    wait_out(0)
    wait_out(1)


def _sc_sum(tensors):
    mesh = plsc.VectorSubcoreMesh(core_axis_name="c", subcore_axis_name="s")
    k = functools.partial(
        pl.kernel,
        mesh=mesh,
        out_type=jax.ShapeDtypeStruct((SC_ROWS, DIM), jnp.float32),
        scratch_types=[
            pltpu.VMEM((2, 8, CHUNK_ROWS, DIM), jnp.float32),
            pltpu.VMEM((2, CHUNK_ROWS, DIM), jnp.float32),
            pltpu.SemaphoreType.DMA((2,)),
            pltpu.SemaphoreType.DMA((2,)),
        ],
    )(_sc_body)
    return k(*tensors)


def _tc_sum8_body(t0, t1, t2, t3, t4, t5, t6, t7, o):
    o[...] = (((t0[...] + t1[...]) + (t2[...] + t3[...]))
              + ((t4[...] + t5[...]) + (t6[...] + t7[...])))


def _tc_sum(tensors):
    off = SC_ROWS // TC_BLOCK_ROWS
    in_spec = pl.BlockSpec((TC_BLOCK_ROWS, DIM), lambda i: (i + off, 0))
    out_spec = pl.BlockSpec((TC_BLOCK_ROWS, DIM), lambda i: (i, 0))
    return pl.pallas_call(
        _tc_sum8_body,
        grid=(TC_ROWS // TC_BLOCK_ROWS,),
        in_specs=[in_spec] * 8,
        out_specs=out_spec,
        out_shape=jax.ShapeDtypeStruct((TC_ROWS, DIM), jnp.float32),
    )(*tensors)


def kernel(tensors_0, tensors_1, tensors_2, tensors_3, tensors_4, tensors_5, tensors_6, tensors_7):
    tensors = (tensors_0, tensors_1, tensors_2, tensors_3,
               tensors_4, tensors_5, tensors_6, tensors_7)
    sc_part = _sc_sum(tensors)
    tc_part = _tc_sum(tensors)
    return jnp.concatenate([sc_part, tc_part], axis=0)


# speedup vs baseline: 2.9756x; 1.1997x over previous
import functools

import jax
import jax.numpy as jnp
from jax import lax
from jax.experimental import pallas as pl
from jax.experimental.pallas import tpu as pltpu
from jax.experimental.pallas import tpu_sc as plsc

BATCH = 4096
DIM = 3328

SC_ROWS = 1280
TC_ROWS = BATCH - SC_ROWS

NUM_WORKERS = 32
ROWS_PER_WORKER = SC_ROWS // NUM_WORKERS
CHUNK_ROWS = 2
NUM_CHUNKS = ROWS_PER_WORKER // CHUNK_ROWS
VREGS_PER_ROW = DIM // 16

TC_BLOCK_ROWS = 256


def _sc_body(t0, t1, t2, t3, t4, t5, t6, t7, out,
             bufs, obuf, in_sems, out_sems):
    tensors = (t0, t1, t2, t3, t4, t5, t6, t7)
    wid = lax.axis_index("s") * 2 + lax.axis_index("c")
    row0 = wid * ROWS_PER_WORKER

    def issue_in(c, b):
        r0 = row0 + c * CHUNK_ROWS
        for k in range(8):
            pltpu.async_copy(
                tensors[k].at[pl.ds(r0, CHUNK_ROWS), :],
                bufs.at[b, k], in_sems.at[b])

    def wait_in(b):
        for k in range(8):
            pltpu.make_async_copy(
                tensors[k].at[pl.ds(0, CHUNK_ROWS), :],
                bufs.at[b, k], in_sems.at[b]).wait()

    def wait_out(b):
        pltpu.make_async_copy(
            obuf.at[b], out.at[pl.ds(0, CHUNK_ROWS), :],
            out_sems.at[b]).wait()

    issue_in(0, 0)

    def outer(ci, _):
        for b in range(2):
            c = ci + b

            @pl.when(c + 1 < NUM_CHUNKS)
            def _():
                issue_in(c + 1, 1 - b)

            wait_in(b)

            @pl.when(c >= 2)
            def _():
                wait_out(b)

            def add_body(j, _):
                sl = pl.ds(j * 16, 16)
                for r in range(CHUNK_ROWS):
                    s = ((bufs[b, 0, r, sl] + bufs[b, 1, r, sl])
                         + (bufs[b, 2, r, sl] + bufs[b, 3, r, sl])) + \
                        ((bufs[b, 4, r, sl] + bufs[b, 5, r, sl])
                         + (bufs[b, 6, r, sl] + bufs[b, 7, r, sl]))
                    obuf[b, r, sl] = s
                return 0

            lax.fori_loop(0, VREGS_PER_ROW, add_body, 0, unroll=2)

            r0 = row0 + c * CHUNK_ROWS
            pltpu.async_copy(
                obuf.at[b], out.at[pl.ds(r0, CHUNK_ROWS), :],
                out_sems.at[b])
        return 0

    lax.fori_loop(0, NUM_CHUNKS // 2, lambda i, x: outer(i * 2, x), 0)
    wait_out(0)
    wait_out(1)


def _sc_sum(tensors):
    mesh = plsc.VectorSubcoreMesh(core_axis_name="c", subcore_axis_name="s")
    k = functools.partial(
        pl.kernel,
        mesh=mesh,
        out_type=jax.ShapeDtypeStruct((SC_ROWS, DIM), jnp.float32),
        scratch_types=[
            pltpu.VMEM((2, 8, CHUNK_ROWS, DIM), jnp.float32),
            pltpu.VMEM((2, CHUNK_ROWS, DIM), jnp.float32),
            pltpu.SemaphoreType.DMA((2,)),
            pltpu.SemaphoreType.DMA((2,)),
        ],
    )(_sc_body)
    return k(*tensors)


def _tc_sum8_body(t0, t1, t2, t3, t4, t5, t6, t7, o):
    o[...] = (((t0[...] + t1[...]) + (t2[...] + t3[...]))
              + ((t4[...] + t5[...]) + (t6[...] + t7[...])))


def _tc_sum(tensors):
    off = SC_ROWS // TC_BLOCK_ROWS
    in_spec = pl.BlockSpec((TC_BLOCK_ROWS, DIM), lambda i: (i + off, 0))
    out_spec = pl.BlockSpec((TC_BLOCK_ROWS, DIM), lambda i: (i, 0))
    return pl.pallas_call(
        _tc_sum8_body,
        grid=(TC_ROWS // TC_BLOCK_ROWS,),
        in_specs=[in_spec] * 8,
        out_specs=out_spec,
        out_shape=jax.ShapeDtypeStruct((TC_ROWS, DIM), jnp.float32),
    )(*tensors)


def kernel(tensors_0, tensors_1, tensors_2, tensors_3, tensors_4, tensors_5, tensors_6, tensors_7):
    tensors = (tensors_0, tensors_1, tensors_2, tensors_3,
               tensors_4, tensors_5, tensors_6, tensors_7)
    sc_part = _sc_sum(tensors)
    tc_part = _tc_sum(tensors)
    return (sc_part, tc_part)


# speedup vs baseline: 3.4336x; 1.1539x over previous
import jax
import jax.numpy as jnp
from jax.experimental import pallas as pl

BATCH = 4096
DIM = 3328
BLOCK_ROWS = 128


def _sum8_kernel(t0, t1, t2, t3, t4, t5, t6, t7, o):
    o[...] = (((t0[...] + t1[...]) + (t2[...] + t3[...]))
              + ((t4[...] + t5[...]) + (t6[...] + t7[...])))


def kernel(tensors_0, tensors_1, tensors_2, tensors_3, tensors_4, tensors_5, tensors_6, tensors_7):
    spec = pl.BlockSpec((BLOCK_ROWS, DIM), lambda i: (i, 0))
    return pl.pallas_call(
        _sum8_kernel,
        grid=(BATCH // BLOCK_ROWS,),
        in_specs=[spec] * 8,
        out_specs=spec,
        out_shape=jax.ShapeDtypeStruct((BATCH, DIM), jnp.float32),
    )(tensors_0, tensors_1, tensors_2, tensors_3,
      tensors_4, tensors_5, tensors_6, tensors_7)


# speedup vs baseline: 3.4362x; 1.0008x over previous
import jax
import jax.numpy as jnp
from jax.experimental import pallas as pl

BATCH = 4096
DIM = 3328
BLOCK_ROWS = 64


def _sum8_kernel(t0, t1, t2, t3, t4, t5, t6, t7, o):
    o[...] = (((t0[...] + t1[...]) + (t2[...] + t3[...]))
              + ((t4[...] + t5[...]) + (t6[...] + t7[...])))


def kernel(tensors_0, tensors_1, tensors_2, tensors_3, tensors_4, tensors_5, tensors_6, tensors_7):
    spec = pl.BlockSpec((BLOCK_ROWS, DIM), lambda i: (i, 0))
    return pl.pallas_call(
        _sum8_kernel,
        grid=(BATCH // BLOCK_ROWS,),
        in_specs=[spec] * 8,
        out_specs=spec,
        out_shape=jax.ShapeDtypeStruct((BATCH, DIM), jnp.float32),
    )(tensors_0, tensors_1, tensors_2, tensors_3,
      tensors_4, tensors_5, tensors_6, tensors_7)
